# baseline (speedup 1.0000x reference)
import functools

import jax
import jax.numpy as jnp
from jax import lax
from jax.experimental import pallas as pl
from jax.experimental.pallas import tpu as pltpu
from jax.experimental.pallas import tpu_sc as plsc

N_PAD = 10240
ROWS_PER_TILE = N_PAD // 16
DUMMY = 10200
CHUNK = 128
CH_SPLIT = 80
E_PAD = 32 * CH_SPLIT * CHUNK


def _fill(buf, rows, f, value):
    vec = jnp.full((16,), value, jnp.float32)

    def body(i, _):
        for j in range(f // 16):
            buf[i, pl.ds(j * 16, 16)] = vec
        return 0

    lax.fori_loop(0, rows, body, 0)


def _zero_slab(zbuf, acc, r0):
    for k in range(ROWS_PER_TILE // 128):
        pltpu.sync_copy(zbuf, acc.at[pl.ds(r0 + k * 128, 128)])


def _edge_pipeline(g_hbm, acc, src_v, dst_v, bufs, gsems, ssems, n_chunks):
    A0, A1, B0, B1 = bufs
    ga0, ga1, gb0, gb1 = gsems
    sa0, sa1, sb0, sb1 = ssems
    ng = n_chunks // 4

    pltpu.async_copy(g_hbm.at[src_v.at[0]], A0, ga0)
    pltpu.async_copy(g_hbm.at[src_v.at[1]], A1, ga1)

    def body(g, _):
        i = 4 * g
        pltpu.make_async_copy(g_hbm.at[src_v.at[i]], A0, ga0).wait()
        pltpu.make_async_copy(g_hbm.at[src_v.at[i + 1]], A1, ga1).wait()

        @pl.when(g > 0)
        def _():
            pltpu.make_async_copy(B0, acc.at[dst_v.at[i - 2]], sb0).wait()
            pltpu.make_async_copy(B1, acc.at[dst_v.at[i - 1]], sb1).wait()

        pltpu.async_copy(g_hbm.at[src_v.at[i + 2]], B0, gb0)
        pltpu.async_copy(g_hbm.at[src_v.at[i + 3]], B1, gb1)
        pltpu.async_copy(A0, acc.at[dst_v.at[i]], sa0, add=True)
        pltpu.async_copy(A1, acc.at[dst_v.at[i + 1]], sa1, add=True)

        pltpu.make_async_copy(g_hbm.at[src_v.at[i + 2]], B0, gb0).wait()
        pltpu.make_async_copy(g_hbm.at[src_v.at[i + 3]], B1, gb1).wait()
        pltpu.make_async_copy(A0, acc.at[dst_v.at[i]], sa0).wait()
        pltpu.make_async_copy(A1, acc.at[dst_v.at[i + 1]], sa1).wait()

        @pl.when(g + 1 < ng)
        def _():
            pltpu.async_copy(g_hbm.at[src_v.at[i + 4]], A0, ga0)
            pltpu.async_copy(g_hbm.at[src_v.at[i + 5]], A1, ga1)

        pltpu.async_copy(B0, acc.at[dst_v.at[i + 2]], sb0, add=True)
        pltpu.async_copy(B1, acc.at[dst_v.at[i + 3]], sb1, add=True)
        return 0

    lax.fori_loop(0, ng, body, 0)

    pltpu.make_async_copy(B0, acc.at[dst_v.at[n_chunks - 2]], sb0).wait()
    pltpu.make_async_copy(B1, acc.at[dst_v.at[n_chunks - 1]], sb1).wait()


def _edge_pipeline2(g_hbm, acc, src_v, dst_v, bufs, gsems, ssems, n_chunks):
    A, B = bufs
    ga, gb = gsems
    sa, sb = ssems
    ng = n_chunks // 2

    pltpu.async_copy(g_hbm.at[src_v.at[0]], A, ga)

    def body(p, _):
        i = 2 * p
        pltpu.make_async_copy(g_hbm.at[src_v.at[i]], A, ga).wait()

        @pl.when(p > 0)
        def _():
            pltpu.make_async_copy(B, acc.at[dst_v.at[i - 1]], sb).wait()

        pltpu.async_copy(g_hbm.at[src_v.at[i + 1]], B, gb)
        pltpu.async_copy(A, acc.at[dst_v.at[i]], sa, add=True)

        pltpu.make_async_copy(g_hbm.at[src_v.at[i + 1]], B, gb).wait()
        pltpu.make_async_copy(A, acc.at[dst_v.at[i]], sa).wait()

        @pl.when(p + 1 < ng)
        def _():
            pltpu.async_copy(g_hbm.at[src_v.at[i + 2]], A, ga)

        pltpu.async_copy(B, acc.at[dst_v.at[i + 1]], sb, add=True)
        return 0

    lax.fori_loop(0, ng, body, 0)

    pltpu.make_async_copy(B, acc.at[dst_v.at[n_chunks - 1]], sb).wait()


def _make_prop_wide():
    mesh = plsc.VectorSubcoreMesh(core_axis_name="c", subcore_axis_name="s")
    half = CH_SPLIT // 2

    @functools.partial(
        pl.kernel,
        out_type=jax.ShapeDtypeStruct((2, N_PAD, 128), jnp.float32),
        mesh=mesh,
        compiler_params=pltpu.CompilerParams(use_tc_tiling_on_sc=False),
        scratch_types=[
            pltpu.VMEM((half, CHUNK), jnp.int32),
            pltpu.VMEM((half, CHUNK), jnp.int32),
            pltpu.VMEM((CHUNK, 128), jnp.float32),
            pltpu.VMEM((CHUNK, 128), jnp.float32),
            pltpu.VMEM_SHARED((N_PAD, 128), jnp.float32),
        ] + [pltpu.SemaphoreType.DMA] * 4,
    )
    def prop(g_hbm, src_hbm, dst_hbm, out_hbm,
             src_v, dst_v, a, b, acc, ga, gb, sa, sb):
        c = lax.axis_index("c")
        s = lax.axis_index("s")
        wid = s * 2 + c

        _fill(a, CHUNK, 128, 0.0)
        r0 = s * ROWS_PER_TILE
        _zero_slab(a, acc, r0)
        plsc.subcore_barrier()

        for h in range(2):
            pltpu.sync_copy(src_hbm.at[wid, pl.ds(h * half, half)], src_v)
            pltpu.sync_copy(dst_hbm.at[wid, pl.ds(h * half, half)], dst_v)
            _edge_pipeline2(g_hbm, acc, src_v, dst_v,
                            (a, b), (ga, gb), (sa, sb), half)

        plsc.subcore_barrier()
        pltpu.sync_copy(acc.at[pl.ds(r0, ROWS_PER_TILE)],
                        out_hbm.at[c, pl.ds(r0, ROWS_PER_TILE)])

    return prop


def _make_prop16():
    mesh = plsc.VectorSubcoreMesh(core_axis_name="c", subcore_axis_name="s")

    @functools.partial(
        pl.kernel,
        out_type=jax.ShapeDtypeStruct((2, N_PAD, 16), jnp.float32),
        mesh=mesh,
        compiler_params=pltpu.CompilerParams(use_tc_tiling_on_sc=False),
        scratch_types=[
            pltpu.VMEM((CH_SPLIT, CHUNK), jnp.int32),
            pltpu.VMEM((CH_SPLIT, CHUNK), jnp.int32),
            pltpu.VMEM((CHUNK, 16), jnp.float32),
            pltpu.VMEM((CHUNK, 16), jnp.float32),
            pltpu.VMEM((CHUNK, 16), jnp.float32),
            pltpu.VMEM((CHUNK, 16), jnp.float32),
            pltpu.VMEM_SHARED((N_PAD, 16), jnp.float32),
        ] + [pltpu.SemaphoreType.DMA] * 8,
    )
    def prop(g_hbm, src_hbm, dst_hbm, out_hbm,
             src_v, dst_v, a0, a1, b0, b1, acc,
             ga0, ga1, gb0, gb1, sa0, sa1, sb0, sb1):
        c = lax.axis_index("c")
        s = lax.axis_index("s")
        wid = s * 2 + c

        pltpu.sync_copy(src_hbm.at[wid], src_v)
        pltpu.sync_copy(dst_hbm.at[wid], dst_v)

        _fill(a0, CHUNK, 16, 0.0)
        r0 = s * ROWS_PER_TILE
        _zero_slab(a0, acc, r0)
        plsc.subcore_barrier()

        _edge_pipeline(g_hbm, acc, src_v, dst_v,
                       (a0, a1, b0, b1), (ga0, ga1, gb0, gb1),
                       (sa0, sa1, sb0, sb1), CH_SPLIT)

        plsc.subcore_barrier()
        pltpu.sync_copy(acc.at[pl.ds(r0, ROWS_PER_TILE)],
                        out_hbm.at[c, pl.ds(r0, ROWS_PER_TILE)])

    return prop


def _make_degrees():
    mesh = plsc.VectorSubcoreMesh(core_axis_name="c", subcore_axis_name="s")

    @functools.partial(
        pl.kernel,
        out_type=jax.ShapeDtypeStruct((2, 2, N_PAD, 16), jnp.float32),
        mesh=mesh,
        compiler_params=pltpu.CompilerParams(use_tc_tiling_on_sc=False),
        scratch_types=[
            pltpu.VMEM((CH_SPLIT, CHUNK), jnp.int32),
            pltpu.VMEM((CH_SPLIT, CHUNK), jnp.int32),
            pltpu.VMEM((CHUNK, 16), jnp.float32),
            pltpu.VMEM_SHARED((N_PAD, 16), jnp.float32),
            pltpu.VMEM_SHARED((N_PAD, 16), jnp.float32),
        ] + [pltpu.SemaphoreType.DMA] * 4,
    )
    def degrees(src_hbm, dst_hbm, out_hbm, src_v, dst_v, ones_v,
                acc_o, acc_i, so0, so1, si0, si1):
        c = lax.axis_index("c")
        s = lax.axis_index("s")
        wid = s * 2 + c

        pltpu.sync_copy(src_hbm.at[wid], src_v)
        pltpu.sync_copy(dst_hbm.at[wid], dst_v)

        _fill(ones_v, CHUNK, 16, 0.0)
        r0 = s * ROWS_PER_TILE
        _zero_slab(ones_v, acc_o, r0)
        _zero_slab(ones_v, acc_i, r0)
        _fill(ones_v, CHUNK, 16, 1.0)
        plsc.subcore_barrier()

        def step(p, _):
            i = 2 * p

            @pl.when(p > 0)
            def _():
                pltpu.make_async_copy(ones_v, acc_o.at[src_v.at[i - 2]], so0).wait()
                pltpu.make_async_copy(ones_v, acc_i.at[dst_v.at[i - 2]], si0).wait()
                pltpu.make_async_copy(ones_v, acc_o.at[src_v.at[i - 1]], so1).wait()
                pltpu.make_async_copy(ones_v, acc_i.at[dst_v.at[i - 1]], si1).wait()

            pltpu.async_copy(ones_v, acc_o.at[src_v.at[i]], so0, add=True)
            pltpu.async_copy(ones_v, acc_i.at[dst_v.at[i]], si0, add=True)
            pltpu.async_copy(ones_v, acc_o.at[src_v.at[i + 1]], so1, add=True)
            pltpu.async_copy(ones_v, acc_i.at[dst_v.at[i + 1]], si1, add=True)
            return 0

        lax.fori_loop(0, CH_SPLIT // 2, step, 0)

        last = CH_SPLIT - 2
        pltpu.make_async_copy(ones_v, acc_o.at[src_v.at[last]], so0).wait()
        pltpu.make_async_copy(ones_v, acc_i.at[dst_v.at[last]], si0).wait()
        pltpu.make_async_copy(ones_v, acc_o.at[src_v.at[last + 1]], so1).wait()
        pltpu.make_async_copy(ones_v, acc_i.at[dst_v.at[last + 1]], si1).wait()

        plsc.subcore_barrier()
        pltpu.sync_copy(acc_o.at[pl.ds(r0, ROWS_PER_TILE)],
                        out_hbm.at[c, 0, pl.ds(r0, ROWS_PER_TILE)])
        pltpu.sync_copy(acc_i.at[pl.ds(r0, ROWS_PER_TILE)],
                        out_hbm.at[c, 1, pl.ds(r0, ROWS_PER_TILE)])

    return degrees


_prop_wide = _make_prop_wide()
_prop16 = _make_prop16()
_degrees = _make_degrees()


def _norms(degp_ref):
    deg_o = degp_ref[0, 0] + degp_ref[1, 0]
    deg_i = degp_ref[0, 1] + degp_ref[1, 1]
    no = lax.rsqrt(jnp.maximum(deg_o[:, 0:1], 1.0))
    ni = lax.rsqrt(jnp.maximum(deg_i[:, 0:1], 1.0))
    return no, ni


def _t1_body(x_ref, degp_ref, w_ref, g_ref):
    no, _ = _norms(degp_ref)
    g_ref[...] = jnp.dot(x_ref[...] * no, w_ref[...],
                         preferred_element_type=jnp.float32)


def _tmid_body(aggp_ref, degp_ref, b_ref, w_ref, g_ref):
    no, ni = _norms(degp_ref)
    agg = aggp_ref[0] + aggp_ref[1]
    h = jnp.maximum(agg * ni + b_ref[...], 0.0)
    g_ref[...] = jnp.dot(h * no, w_ref[...], preferred_element_type=jnp.float32)


def _t4_body(aggp_ref, degp_ref, b_ref, g_ref):
    no, ni = _norms(degp_ref)
    agg = aggp_ref[0] + aggp_ref[1]
    h = jnp.maximum(agg * ni + b_ref[...], 0.0)
    g_ref[...] = h * no


def _t5_body(aggp_ref, degp_ref, w_ref, b_ref, out_ref):
    _, ni = _norms(degp_ref)
    agg = aggp_ref[0] + aggp_ref[1]
    out_ref[...] = (jnp.dot(agg * ni, w_ref[...],
                            preferred_element_type=jnp.float32) + b_ref[...])


def _tc(body, out_shape, *args):
    return pl.pallas_call(
        body,
        out_shape=jax.ShapeDtypeStruct(out_shape, jnp.float32),
    )(*args)


def kernel(x, edge_index, W1, b1, W2, b2, W3, b3, W4, b4):
    n = x.shape[0]
    e = edge_index.shape[1]
    src = edge_index[0].astype(jnp.int32)
    dst = edge_index[1].astype(jnp.int32)
    pad = jnp.full((E_PAD - e,), DUMMY, jnp.int32)
    src_p = jnp.concatenate([src, pad])
    dst_p = jnp.concatenate([dst, pad])
    src32 = src_p.reshape(32, CH_SPLIT, CHUNK)
    dst32 = dst_p.reshape(32, CH_SPLIT, CHUNK)
    x_pad = jnp.pad(x, ((0, N_PAD - n), (0, 0)))
    w3p = jnp.pad(W3, ((0, 0), (0, 16 - W3.shape[1])))
    b3p = jnp.pad(b3, (0, 16 - b3.shape[0])).reshape(1, 16)
    w4p = jnp.pad(W4, ((0, 16 - W4.shape[0]), (0, 0)))
    b1r = b1.reshape(1, -1)
    b2r = b2.reshape(1, -1)
    b4r = b4.reshape(1, -1)

    degp = _degrees(src32, dst32)

    g1 = _tc(_t1_body, (N_PAD, 128), x_pad, degp, W1)
    a1 = _prop_wide(g1, src32, dst32)
    g2 = _tc(_tmid_body, (N_PAD, 128), a1, degp, b1r, W2)
    a2 = _prop_wide(g2, src32, dst32)
    g3 = _tc(_tmid_body, (N_PAD, 16), a2, degp, b2r, w3p)
    a3 = _prop16(g3, src32, dst32)
    g4 = _tc(_t4_body, (N_PAD, 16), a3, degp, b3p)
    a4 = _prop16(g4, src32, dst32)
    out = _tc(_t5_body, (N_PAD, 16), a4, degp, w4p, b4r)

    return out[:n]


# speedup vs baseline: 1.2669x; 1.2669x over previous
import functools

import jax
import jax.numpy as jnp
from jax import lax
from jax.experimental import pallas as pl
from jax.experimental.pallas import tpu as pltpu
from jax.experimental.pallas import tpu_sc as plsc

N_PAD = 10240
ROWS_PER_TILE = N_PAD // 16
DUMMY = 10200
CHUNK = 128
CH_SPLIT = 80
CH_FULL = 160
E_PAD = 32 * CH_SPLIT * CHUNK
K = 4


def _fill(buf, rows, f, value):
    vec = jnp.full((16,), value, jnp.float32)

    def body(i, _):
        for j in range(f // 16):
            buf[i, pl.ds(j * 16, 16)] = vec
        return 0

    lax.fori_loop(0, rows, body, 0)


def _zero_slab(zbuf, acc, r0):
    for k in range(ROWS_PER_TILE // 128):
        pltpu.sync_copy(zbuf, acc.at[pl.ds(r0 + k * 128, 128)])


def _pipeline(g_hbm, acc, src_v, dst_v, A, B, gA, gB, sA, sB, n_chunks):
    ng = n_chunks // (2 * K)
    for k in range(K):
        pltpu.async_copy(g_hbm.at[src_v.at[k]], A[k], gA[k])

    def body(g, _):
        i = 2 * K * g
        for k in range(K):
            pltpu.make_async_copy(g_hbm.at[src_v.at[i + k]], A[k], gA[k]).wait()

        @pl.when(g > 0)
        def _():
            for k in range(K):
                pltpu.make_async_copy(
                    B[k], acc.at[dst_v.at[i - K + k]], sB[k]).wait()

        for k in range(K):
            pltpu.async_copy(g_hbm.at[src_v.at[i + K + k]], B[k], gB[k])
        for k in range(K):
            pltpu.async_copy(A[k], acc.at[dst_v.at[i + k]], sA[k], add=True)

        for k in range(K):
            pltpu.make_async_copy(
                g_hbm.at[src_v.at[i + K + k]], B[k], gB[k]).wait()
        for k in range(K):
            pltpu.make_async_copy(A[k], acc.at[dst_v.at[i + k]], sA[k]).wait()

        @pl.when(g + 1 < ng)
        def _():
            for k in range(K):
                pltpu.async_copy(g_hbm.at[src_v.at[i + 2 * K + k]], A[k], gA[k])

        for k in range(K):
            pltpu.async_copy(B[k], acc.at[dst_v.at[i + K + k]], sB[k], add=True)
        return 0

    lax.fori_loop(0, ng, body, 0)

    for k in range(K):
        pltpu.make_async_copy(
            B[k], acc.at[dst_v.at[n_chunks - K + k]], sB[k]).wait()


_SEMS = [pltpu.SemaphoreType.DMA] * (4 * K)


def _make_prop_wide():
    mesh = plsc.VectorSubcoreMesh(core_axis_name="c", subcore_axis_name="s")
    half = CH_FULL // 2

    @functools.partial(
        pl.kernel,
        out_type=jax.ShapeDtypeStruct((2 * N_PAD, 64), jnp.float32),
        mesh=mesh,
        compiler_params=pltpu.CompilerParams(use_tc_tiling_on_sc=False),
        scratch_types=[
            pltpu.VMEM((half, CHUNK), jnp.int32),
            pltpu.VMEM((half, CHUNK), jnp.int32),
        ] + [pltpu.VMEM((CHUNK, 64), jnp.float32)] * (2 * K) + [
            pltpu.VMEM_SHARED((N_PAD, 64), jnp.float32),
        ] + _SEMS,
    )
    def prop(g_hbm, src0_hbm, src1_hbm, dst_hbm, out_hbm,
             src_v, dst_v, *rest):
        bufs, acc, sems = rest[:2 * K], rest[2 * K], rest[2 * K + 1:]
        A, B = bufs[:K], bufs[K:]
        gA, gB = sems[:K], sems[K:2 * K]
        sA, sB = sems[2 * K:3 * K], sems[3 * K:]
        c = lax.axis_index("c")
        s = lax.axis_index("s")

        _fill(A[0], CHUNK, 64, 0.0)
        r0 = s * ROWS_PER_TILE
        _zero_slab(A[0], acc, r0)
        plsc.subcore_barrier()

        for h in range(2):
            @pl.when(c == 0)
            def _():
                pltpu.sync_copy(src0_hbm.at[s, pl.ds(h * half, half)], src_v)

            @pl.when(c == 1)
            def _():
                pltpu.sync_copy(src1_hbm.at[s, pl.ds(h * half, half)], src_v)

            pltpu.sync_copy(dst_hbm.at[s, pl.ds(h * half, half)], dst_v)
            _pipeline(g_hbm, acc, src_v, dst_v, A, B, gA, gB, sA, sB, half)

        plsc.subcore_barrier()
        pltpu.sync_copy(acc.at[pl.ds(r0, ROWS_PER_TILE)],
                        out_hbm.at[pl.ds(c * N_PAD + r0, ROWS_PER_TILE)])

    return prop


def _make_prop16():
    mesh = plsc.VectorSubcoreMesh(core_axis_name="c", subcore_axis_name="s")

    @functools.partial(
        pl.kernel,
        out_type=jax.ShapeDtypeStruct((2, N_PAD, 16), jnp.float32),
        mesh=mesh,
        compiler_params=pltpu.CompilerParams(use_tc_tiling_on_sc=False),
        scratch_types=[
            pltpu.VMEM((CH_SPLIT, CHUNK), jnp.int32),
            pltpu.VMEM((CH_SPLIT, CHUNK), jnp.int32),
        ] + [pltpu.VMEM((CHUNK, 16), jnp.float32)] * (2 * K) + [
            pltpu.VMEM_SHARED((N_PAD, 16), jnp.float32),
        ] + _SEMS,
    )
    def prop(g_hbm, src_hbm, dst_hbm, out_hbm, src_v, dst_v, *rest):
        bufs, acc, sems = rest[:2 * K], rest[2 * K], rest[2 * K + 1:]
        A, B = bufs[:K], bufs[K:]
        gA, gB = sems[:K], sems[K:2 * K]
        sA, sB = sems[2 * K:3 * K], sems[3 * K:]
        c = lax.axis_index("c")
        s = lax.axis_index("s")
        wid = s * 2 + c

        pltpu.sync_copy(src_hbm.at[wid], src_v)
        pltpu.sync_copy(dst_hbm.at[wid], dst_v)

        _fill(A[0], CHUNK, 16, 0.0)
        r0 = s * ROWS_PER_TILE
        _zero_slab(A[0], acc, r0)
        plsc.subcore_barrier()

        _pipeline(g_hbm, acc, src_v, dst_v, A, B, gA, gB, sA, sB, CH_SPLIT)

        plsc.subcore_barrier()
        pltpu.sync_copy(acc.at[pl.ds(r0, ROWS_PER_TILE)],
                        out_hbm.at[c, pl.ds(r0, ROWS_PER_TILE)])

    return prop


def _make_degrees():
    mesh = plsc.VectorSubcoreMesh(core_axis_name="c", subcore_axis_name="s")

    @functools.partial(
        pl.kernel,
        out_type=jax.ShapeDtypeStruct((2, 2, N_PAD, 16), jnp.float32),
        mesh=mesh,
        compiler_params=pltpu.CompilerParams(use_tc_tiling_on_sc=False),
        scratch_types=[
            pltpu.VMEM((CH_SPLIT, CHUNK), jnp.int32),
            pltpu.VMEM((CH_SPLIT, CHUNK), jnp.int32),
            pltpu.VMEM((CHUNK, 16), jnp.float32),
            pltpu.VMEM_SHARED((N_PAD, 16), jnp.float32),
            pltpu.VMEM_SHARED((N_PAD, 16), jnp.float32),
        ] + [pltpu.SemaphoreType.DMA] * 4,
    )
    def degrees(src_hbm, dst_hbm, out_hbm, src_v, dst_v, ones_v,
                acc_o, acc_i, so0, so1, si0, si1):
        c = lax.axis_index("c")
        s = lax.axis_index("s")
        wid = s * 2 + c

        pltpu.sync_copy(src_hbm.at[wid], src_v)
        pltpu.sync_copy(dst_hbm.at[wid], dst_v)

        _fill(ones_v, CHUNK, 16, 0.0)
        r0 = s * ROWS_PER_TILE
        _zero_slab(ones_v, acc_o, r0)
        _zero_slab(ones_v, acc_i, r0)
        _fill(ones_v, CHUNK, 16, 1.0)
        plsc.subcore_barrier()

        def step(p, _):
            i = 2 * p

            @pl.when(p > 0)
            def _():
                pltpu.make_async_copy(ones_v, acc_o.at[src_v.at[i - 2]], so0).wait()
                pltpu.make_async_copy(ones_v, acc_i.at[dst_v.at[i - 2]], si0).wait()
                pltpu.make_async_copy(ones_v, acc_o.at[src_v.at[i - 1]], so1).wait()
                pltpu.make_async_copy(ones_v, acc_i.at[dst_v.at[i - 1]], si1).wait()

            pltpu.async_copy(ones_v, acc_o.at[src_v.at[i]], so0, add=True)
            pltpu.async_copy(ones_v, acc_i.at[dst_v.at[i]], si0, add=True)
            pltpu.async_copy(ones_v, acc_o.at[src_v.at[i + 1]], so1, add=True)
            pltpu.async_copy(ones_v, acc_i.at[dst_v.at[i + 1]], si1, add=True)
            return 0

        lax.fori_loop(0, CH_SPLIT // 2, step, 0)

        last = CH_SPLIT - 2
        pltpu.make_async_copy(ones_v, acc_o.at[src_v.at[last]], so0).wait()
        pltpu.make_async_copy(ones_v, acc_i.at[dst_v.at[last]], si0).wait()
        pltpu.make_async_copy(ones_v, acc_o.at[src_v.at[last + 1]], so1).wait()
        pltpu.make_async_copy(ones_v, acc_i.at[dst_v.at[last + 1]], si1).wait()

        plsc.subcore_barrier()
        pltpu.sync_copy(acc_o.at[pl.ds(r0, ROWS_PER_TILE)],
                        out_hbm.at[c, 0, pl.ds(r0, ROWS_PER_TILE)])
        pltpu.sync_copy(acc_i.at[pl.ds(r0, ROWS_PER_TILE)],
                        out_hbm.at[c, 1, pl.ds(r0, ROWS_PER_TILE)])

    return degrees


_prop_wide = _make_prop_wide()
_prop16 = _make_prop16()
_degrees = _make_degrees()


def _norms(degp_ref):
    deg_o = degp_ref[0, 0] + degp_ref[1, 0]
    deg_i = degp_ref[0, 1] + degp_ref[1, 1]
    no = lax.rsqrt(jnp.maximum(deg_o[:, 0:1], 1.0))
    ni = lax.rsqrt(jnp.maximum(deg_i[:, 0:1], 1.0))
    return no, ni


def _stack_halves(res):
    return jnp.concatenate([res[:, :64], res[:, 64:]], axis=0)


def _unstack_halves(a):
    return jnp.concatenate([a[:N_PAD], a[N_PAD:]], axis=1)


def _t1_body(x_ref, degp_ref, w_ref, g_ref):
    no, _ = _norms(degp_ref)
    res = jnp.dot(x_ref[...] * no, w_ref[...], preferred_element_type=jnp.float32)
    g_ref[...] = _stack_halves(res)


def _t2_body(a_ref, degp_ref, b_ref, w_ref, g_ref):
    no, ni = _norms(degp_ref)
    agg = _unstack_halves(a_ref[...])
    h = jnp.maximum(agg * ni + b_ref[...], 0.0)
    res = jnp.dot(h * no, w_ref[...], preferred_element_type=jnp.float32)
    g_ref[...] = _stack_halves(res)


def _t3_body(a_ref, degp_ref, b_ref, w_ref, g_ref):
    no, ni = _norms(degp_ref)
    agg = _unstack_halves(a_ref[...])
    h = jnp.maximum(agg * ni + b_ref[...], 0.0)
    g_ref[...] = jnp.dot(h * no, w_ref[...], preferred_element_type=jnp.float32)


def _t4_body(aggp_ref, degp_ref, b_ref, g_ref):
    no, ni = _norms(degp_ref)
    agg = aggp_ref[0] + aggp_ref[1]
    h = jnp.maximum(agg * ni + b_ref[...], 0.0)
    g_ref[...] = h * no


def _t5_body(aggp_ref, degp_ref, w_ref, b_ref, out_ref):
    _, ni = _norms(degp_ref)
    agg = aggp_ref[0] + aggp_ref[1]
    out_ref[...] = (jnp.dot(agg * ni, w_ref[...],
                            preferred_element_type=jnp.float32) + b_ref[...])


def _tc(body, out_shape, *args):
    return pl.pallas_call(
        body,
        out_shape=jax.ShapeDtypeStruct(out_shape, jnp.float32),
    )(*args)


def kernel(x, edge_index, W1, b1, W2, b2, W3, b3, W4, b4):
    n = x.shape[0]
    e = edge_index.shape[1]
    src = edge_index[0].astype(jnp.int32)
    dst = edge_index[1].astype(jnp.int32)
    pad = jnp.full((E_PAD - e,), DUMMY, jnp.int32)
    src_p = jnp.concatenate([src, pad])
    dst_p = jnp.concatenate([dst, pad])
    src32 = src_p.reshape(32, CH_SPLIT, CHUNK)
    dst32 = dst_p.reshape(32, CH_SPLIT, CHUNK)
    src16 = src_p.reshape(16, CH_FULL, CHUNK)
    dst16 = dst_p.reshape(16, CH_FULL, CHUNK)
    src16b = src16 + N_PAD

    x_pad = jnp.pad(x, ((0, N_PAD - n), (0, 0)))
    w3p = jnp.pad(W3, ((0, 0), (0, 16 - W3.shape[1])))
    b3p = jnp.pad(b3, (0, 16 - b3.shape[0])).reshape(1, 16)
    w4p = jnp.pad(W4, ((0, 16 - W4.shape[0]), (0, 0)))
    b1r = b1.reshape(1, -1)
    b2r = b2.reshape(1, -1)
    b4r = b4.reshape(1, -1)

    degp = _degrees(src32, dst32)

    g1 = _tc(_t1_body, (2 * N_PAD, 64), x_pad, degp, W1)
    a1 = _prop_wide(g1, src16, src16b, dst16)
    g2 = _tc(_t2_body, (2 * N_PAD, 64), a1, degp, b1r, W2)
    a2 = _prop_wide(g2, src16, src16b, dst16)
    g3 = _tc(_t3_body, (N_PAD, 16), a2, degp, b2r, w3p)
    a3 = _prop16(g3, src32, dst32)
    g4 = _tc(_t4_body, (N_PAD, 16), a3, degp, b3p)
    a4 = _prop16(g4, src32, dst32)
    out = _tc(_t5_body, (N_PAD, 16), a4, degp, w4p, b4r)

    return out[:n]
